# baseline (speedup 1.0000x reference)
import functools

import jax
import jax.numpy as jnp
from jax import lax
from jax.experimental import pallas as pl
from jax.experimental.pallas import tpu as pltpu
from jax.experimental.pallas import tpu_sc as plsc

B, L, K, AA = 8, 512, 30, 20
S = B * L
BLK = AA * AA

NC, NS, LANES = 2, 16, 16

NW_E = NS
SPW_E = S // NW_E


def _eaa_body(eidx_hbm, ref_hbm, out_hbm, eidx_v, ref_v, out_v, sem):
    wid = lax.axis_index("s")
    wbase = wid * SPW_E
    b = wid // (L // SPW_E)
    l0 = (wid % (L // SPW_E)) * SPW_E
    b512 = b * L
    pltpu.sync_copy(eidx_hbm.at[pl.ds(wbase * K, SPW_E * K)], eidx_v)
    pltpu.sync_copy(ref_hbm, ref_v)

    lane = lax.iota(jnp.int32, LANES)

    def kbody(k, _):
        def gbody(g, _):
            sl = g * LANES + lane
            eidx = plsc.load_gather(eidx_v, [sl * K + k])
            out_v[k, pl.ds(g * LANES, LANES)] = plsc.load_gather(
                ref_v, [eidx + b512])
            return 0
        lax.fori_loop(0, SPW_E // LANES, gbody, 0)
        return 0

    lax.fori_loop(0, K, kbody, 0)
    pltpu.sync_copy(out_v, out_hbm.at[b, :, pl.ds(l0, SPW_E)])


_EAA_CACHE = []


def _sc_eaa(*args):
    if not _EAA_CACHE:
        _EAA_CACHE.append(functools.partial(
            pl.kernel,
            out_type=jax.ShapeDtypeStruct((B, K, L), jnp.int32),
            mesh=plsc.VectorSubcoreMesh(core_axis_name="c",
                                        subcore_axis_name="s",
                                        num_cores=1, num_subcores=NS),
            scratch_types=[
                pltpu.VMEM((SPW_E * K,), jnp.int32),
                pltpu.VMEM((S,), jnp.int32),
                pltpu.VMEM((K, SPW_E), jnp.int32),
                pltpu.SemaphoreType.DMA,
            ],
            compiler_params=pltpu.CompilerParams(needs_layout_passes=False),
        )(_eaa_body))
    return _EAA_CACHE[0](*args)


KB = 30


def _tc_body(eaa_ref, etab_ref, out_ref):
    kk = pl.program_id(1)
    jrow = lax.broadcasted_iota(jnp.int32, (BLK, L), 0)
    jm = jrow % AA
    diag = jm == jrow // AA
    acc = None
    for dk in range(KB):
        e = etab_ref[0, dk]
        eaa = eaa_ref[0, dk, 0]
        sel = jnp.where(jm == eaa[None, :], e, 0.0)
        if dk == 0:
            sel = jnp.where(kk == 0, jnp.where(diag, e, 0.0), sel)
        acc = sel if dk == 0 else acc + sel
    seg = (lax.broadcasted_iota(jnp.int32, (AA, BLK), 1) // AA ==
           lax.broadcasted_iota(jnp.int32, (AA, BLK), 0)).astype(jnp.float32)
    contrib = jnp.dot(seg, acc, preferred_element_type=jnp.float32)

    @pl.when(kk == 0)
    def _():
        out_ref[...] = contrib[None]

    @pl.when(kk > 0)
    def _():
        out_ref[...] += contrib[None]


def _tc_gather(eaa, etab_t):
    return pl.pallas_call(
        _tc_body,
        grid=(B, K // KB),
        in_specs=[
            pl.BlockSpec((1, KB, 1, L), lambda b, k: (b, k, 0, 0)),
            pl.BlockSpec((1, KB, BLK, L), lambda b, k: (b, k, 0, 0)),
        ],
        out_specs=pl.BlockSpec((1, AA, L), lambda b, k: (b, 0, 0)),
        out_shape=jax.ShapeDtypeStruct((B, AA, L), jnp.float32),
    )(eaa.reshape(B, K, 1, L), etab_t)


def _fin_body(aa_ref, ref_ref, mask_ref, out_ref):
    neg = -aa_ref[...]
    m = jnp.max(neg, axis=1, keepdims=True)
    lse = jnp.log(jnp.sum(jnp.exp(neg - m), axis=1)) + m[:, 0, :]
    r = ref_ref[...]
    sel = lax.broadcasted_iota(jnp.int32, (B, AA, L), 1) == r[:, None, :]
    picked = jnp.sum(jnp.where(sel, neg, 0.0), axis=1)
    mask = mask_ref[...]
    num = jnp.sum((picked - lse) * mask, axis=1, keepdims=True)
    den = jnp.sum(mask, axis=1, keepdims=True)
    out_ref[0, 0] = -jnp.sum(num / den) / B


_finish = pl.pallas_call(
    _fin_body,
    out_shape=jax.ShapeDtypeStruct((1, 1), jnp.float32),
    out_specs=pl.BlockSpec(memory_space=pltpu.SMEM),
)


def kernel(etab, E_idx, ref_seqs, x_mask):
    etab_t = jnp.transpose(etab, (0, 2, 3, 1))
    eaa = _sc_eaa(E_idx.reshape(-1), ref_seqs.reshape(-1))
    aa_nrgs = _tc_gather(eaa, etab_t)
    out = _finish(aa_nrgs, ref_seqs, x_mask)
    return out[0, 0]


# speedup vs baseline: 1.0367x; 1.0367x over previous
import functools

import jax
import jax.numpy as jnp
from jax import lax
from jax.experimental import pallas as pl
from jax.experimental.pallas import tpu as pltpu
from jax.experimental.pallas import tpu_sc as plsc

B, L, K, AA = 8, 512, 30, 20
S = B * L
BLK = AA * AA

NC, NS, LANES = 2, 16, 16

NW_E = NS
SPW_E = S // NW_E


def _eaa_body(eidx_hbm, ref_hbm, out_hbm, eidx_v, ref_v, out_v, sem):
    wid = lax.axis_index("s")
    wbase = wid * SPW_E
    b = wid // (L // SPW_E)
    l0 = (wid % (L // SPW_E)) * SPW_E
    b512 = b * L
    pltpu.sync_copy(eidx_hbm.at[pl.ds(wbase * K, SPW_E * K)], eidx_v)
    pltpu.sync_copy(ref_hbm, ref_v)

    lane = lax.iota(jnp.int32, LANES)

    def kbody(k, _):
        def gbody(g, _):
            sl = g * LANES + lane
            eidx = plsc.load_gather(eidx_v, [sl * K + k])
            out_v[k, pl.ds(g * LANES, LANES)] = plsc.load_gather(
                ref_v, [eidx + b512])
            return 0
        lax.fori_loop(0, SPW_E // LANES, gbody, 0)
        return 0

    lax.fori_loop(0, K, kbody, 0)
    pltpu.sync_copy(out_v, out_hbm.at[b, :, pl.ds(l0, SPW_E)])


_EAA_CACHE = []


def _sc_eaa(*args):
    if not _EAA_CACHE:
        _EAA_CACHE.append(functools.partial(
            pl.kernel,
            out_type=jax.ShapeDtypeStruct((B, K, L), jnp.int32),
            mesh=plsc.VectorSubcoreMesh(core_axis_name="c",
                                        subcore_axis_name="s",
                                        num_cores=1, num_subcores=NS),
            scratch_types=[
                pltpu.VMEM((SPW_E * K,), jnp.int32),
                pltpu.VMEM((S,), jnp.int32),
                pltpu.VMEM((K, SPW_E), jnp.int32),
                pltpu.SemaphoreType.DMA,
            ],
            compiler_params=pltpu.CompilerParams(needs_layout_passes=False),
        )(_eaa_body))
    return _EAA_CACHE[0](*args)


KB = 15


def _tc_body(eaa_ref, etab_ref, out_ref):
    kk = pl.program_id(1)
    jrow = lax.broadcasted_iota(jnp.int32, (BLK, L), 0)
    jm = jrow % AA
    diag = jm == jrow // AA
    acc = None
    for dk in range(KB):
        e = etab_ref[0, dk]
        eaa = eaa_ref[0, dk, 0]
        sel = jnp.where(jm == eaa[None, :], e, 0.0)
        if dk == 0:
            sel = jnp.where(kk == 0, jnp.where(diag, e, 0.0), sel)
        acc = sel if dk == 0 else acc + sel
    seg = (lax.broadcasted_iota(jnp.int32, (AA, BLK), 1) // AA ==
           lax.broadcasted_iota(jnp.int32, (AA, BLK), 0)).astype(jnp.float32)
    contrib = jnp.dot(seg, acc, preferred_element_type=jnp.float32)

    @pl.when(kk == 0)
    def _():
        out_ref[...] = contrib[None]

    @pl.when(kk > 0)
    def _():
        out_ref[...] += contrib[None]


def _tc_gather(eaa, etab_t):
    return pl.pallas_call(
        _tc_body,
        grid=(B, K // KB),
        in_specs=[
            pl.BlockSpec((1, KB, 1, L), lambda b, k: (b, k, 0, 0)),
            pl.BlockSpec((1, KB, BLK, L), lambda b, k: (b, k, 0, 0)),
        ],
        out_specs=pl.BlockSpec((1, AA, L), lambda b, k: (b, 0, 0)),
        out_shape=jax.ShapeDtypeStruct((B, AA, L), jnp.float32),
    )(eaa.reshape(B, K, 1, L), etab_t)


def _fin_body(aa_ref, ref_ref, mask_ref, out_ref):
    neg = -aa_ref[...]
    m = jnp.max(neg, axis=1, keepdims=True)
    lse = jnp.log(jnp.sum(jnp.exp(neg - m), axis=1)) + m[:, 0, :]
    r = ref_ref[...]
    sel = lax.broadcasted_iota(jnp.int32, (B, AA, L), 1) == r[:, None, :]
    picked = jnp.sum(jnp.where(sel, neg, 0.0), axis=1)
    mask = mask_ref[...]
    num = jnp.sum((picked - lse) * mask, axis=1, keepdims=True)
    den = jnp.sum(mask, axis=1, keepdims=True)
    out_ref[0, 0] = -jnp.sum(num / den) / B


_finish = pl.pallas_call(
    _fin_body,
    out_shape=jax.ShapeDtypeStruct((1, 1), jnp.float32),
    out_specs=pl.BlockSpec(memory_space=pltpu.SMEM),
)


def kernel(etab, E_idx, ref_seqs, x_mask):
    etab_t = jnp.transpose(etab, (0, 2, 3, 1))
    eaa = _sc_eaa(E_idx.reshape(-1), ref_seqs.reshape(-1))
    aa_nrgs = _tc_gather(eaa, etab_t)
    out = _finish(aa_nrgs, ref_seqs, x_mask)
    return out[0, 0]
